# Initial kernel scaffold; baseline (speedup 1.0000x reference)
import jax
import jax.numpy as jnp
from jax.experimental import pallas as pl


def kernel(nfeat, efeat, edge_index, timestamp, batch_eids, neg_dst, t_w, W1, W2, gamma, beta, Wp, bp):
    raise NotImplementedError("write your pallas kernel here")


# speedup vs baseline: 1.5957x; 1.5957x over previous
import dataclasses
import functools

import jax
import jax.numpy as jnp
from jax import lax
from jax.experimental import pallas as pl
from jax.experimental.pallas import tpu as pltpu
from jax.experimental.pallas import tpu_sc as plsc

N = 10000
NP = 10240
E = 160000
EP = 163840
DE = 16
TD = 16
H = 128
B = 256

NW = 32
EC = 64
PER_W = EP // NW
CHUNKS = PER_W // EC
ROWS_PER_TILE = NP // 16

@functools.cache
def _mesh():
    return plsc.VectorSubcoreMesh(core_axis_name="c", subcore_axis_name="s",
                                  num_cores=2, num_subcores=16)


def _sc_cp():
    cp = pltpu.CompilerParams()
    if "needs_layout_passes" in pltpu.CompilerParams.__dataclass_fields__:
        cp = dataclasses.replace(cp, needs_layout_passes=False)
    return cp


def _k1_node_mm(nfeat_p, wcat):
    def body(x_ref, w_ref, o_ref):
        o_ref[...] = jnp.dot(x_ref[...], w_ref[...],
                             preferred_element_type=jnp.float32)
    return pl.pallas_call(
        body,
        grid=(8,),
        in_specs=[pl.BlockSpec((NP // 8, 128), lambda i: (i, 0)),
                  pl.BlockSpec((128, 256), lambda i: (0, 0))],
        out_specs=pl.BlockSpec((NP // 8, 256), lambda i: (i, 0)),
        out_shape=jax.ShapeDtypeStruct((NP, 256), jnp.float32),
    )(nfeat_p, wcat)


def _k2_edge_t(ts_col, efeat_p, wef, wtf, tw_row):
    def body(ts_ref, ef_ref, wef_ref, wtf_ref, tw_ref, o_ref):
        te = jnp.cos(ts_ref[...] * tw_ref[...])
        o_ref[...] = (
            jnp.dot(ef_ref[...], wef_ref[...],
                    preferred_element_type=jnp.float32)
            + jnp.dot(te, wtf_ref[...],
                      preferred_element_type=jnp.float32))
    blk = 2048
    return pl.pallas_call(
        body,
        grid=(EP // blk,),
        in_specs=[pl.BlockSpec((blk, 1), lambda i: (i, 0)),
                  pl.BlockSpec((blk, 16), lambda i: (i, 0)),
                  pl.BlockSpec((16, 128), lambda i: (0, 0)),
                  pl.BlockSpec((16, 128), lambda i: (0, 0)),
                  pl.BlockSpec((1, 16), lambda i: (0, 0))],
        out_specs=pl.BlockSpec((blk, 128), lambda i: (i, 0)),
        out_shape=jax.ShapeDtypeStruct((EP, 128), jnp.float32),
    )(ts_col, efeat_p, wef, wtf, tw_row)


def _k3_l0_scatter(src_t, dst_t, src_g, dst_g, efw, bt, zero128):
    f32 = jnp.float32
    outs = (
        jax.ShapeDtypeStruct((2 * NP, 128), f32),
        jax.ShapeDtypeStruct((2 * NP, 128), f32),
        jax.ShapeDtypeStruct((2 * NP, 128), f32),
        jax.ShapeDtypeStruct((2 * NP, 128), f32),
    )

    @functools.partial(
        pl.kernel, mesh=_mesh(), out_type=outs,
        scratch_types=[
            pltpu.VMEM_SHARED((NP, 128), f32),
            pltpu.VMEM((EC, 128), f32),
            pltpu.VMEM((EC,), jnp.int32),
            pltpu.VMEM((EC,), jnp.int32),
            pltpu.SemaphoreType.DMA,
        ],
    )
    def kern(src_t_h, dst_t_h, src_g_h, dst_g_h, efw_h, bt_h, z128_h,
             acc_s_h, efc_s_h, acc_d_h, efc_d_h,
             sh, buf, idx_t, idx_g, sem):
        c = lax.axis_index("c")
        s = lax.axis_index("s")
        wid = c * 16 + s
        base = wid * PER_W

        def zero_and_sync():
            pltpu.sync_copy(z128_h, buf)

            @pl.loop(0, ROWS_PER_TILE // EC)
            def _(k):
                r0 = s * ROWS_PER_TILE + k * EC
                pltpu.sync_copy(buf, sh.at[pl.ds(r0, EC)])
            plsc.subcore_barrier()

        def writeback(o_h):
            plsc.subcore_barrier()

            @pl.loop(0, ROWS_PER_TILE // EC)
            def _(k):
                r0 = s * ROWS_PER_TILE + k * EC
                pltpu.sync_copy(sh.at[pl.ds(r0, EC)], buf)
                pltpu.sync_copy(buf, o_h.at[pl.ds(c * NP + r0, EC)])
            plsc.subcore_barrier()

        def pass_gather(t_h, g_h, o_h):
            zero_and_sync()

            @pl.loop(0, CHUNKS)
            def _(k):
                eb = base + k * EC
                pltpu.sync_copy(t_h.at[pl.ds(eb, EC)], idx_t)
                pltpu.sync_copy(g_h.at[pl.ds(eb, EC)], idx_g)
                pltpu.async_copy(bt_h.at[idx_g], buf, sem).wait()
                pltpu.sync_copy(buf, sh.at[idx_t], add=True)
            writeback(o_h)

        def pass_linear(t_h, o_h):
            zero_and_sync()

            @pl.loop(0, CHUNKS)
            def _(k):
                eb = base + k * EC
                pltpu.sync_copy(t_h.at[pl.ds(eb, EC)], idx_t)
                pltpu.sync_copy(efw_h.at[pl.ds(eb, EC)], buf)
                pltpu.sync_copy(buf, sh.at[idx_t], add=True)
            writeback(o_h)

        pass_gather(src_t_h, dst_g_h, acc_s_h)
        pass_linear(src_t_h, efc_s_h)
        pass_gather(dst_t_h, src_g_h, acc_d_h)
        pass_linear(dst_t_h, efc_d_h)

    return kern(src_t, dst_t, src_g, dst_g, efw, bt, zero128)


def _k4_combine1(a_tbl, acc_s0, acc_s1, efc_s0, efc_s1,
                 acc_d0, acc_d1, efc_d0, efc_d1, w1e):
    def body(a_ref, s0, s1, e0, e1, d0, d1, f0, f1, w_ref, ps_ref, pd_ref):
        a = a_ref[...]
        w = w_ref[...]
        es = e0[...] + e1[...]
        ed = f0[...] + f1[...]
        cs = jnp.maximum(es[:, 16:17], 1.0)
        cd = jnp.maximum(ed[:, 16:17], 1.0)
        num_s = (s0[...] + s1[...]) + jnp.dot(
            es[:, 0:16], w, preferred_element_type=jnp.float32)
        num_d = (d0[...] + d1[...]) + jnp.dot(
            ed[:, 0:16], w, preferred_element_type=jnp.float32)
        ps_ref[...] = a + num_s / cs
        pd_ref[...] = a + num_d / cd
    blk = NP // 8
    spec128 = pl.BlockSpec((blk, 128), lambda i: (i, 0))
    return pl.pallas_call(
        body,
        grid=(8,),
        in_specs=[spec128] * 9 + [pl.BlockSpec((16, 128), lambda i: (0, 0))],
        out_specs=[spec128, spec128],
        out_shape=[jax.ShapeDtypeStruct((NP, 128), jnp.float32),
                   jax.ShapeDtypeStruct((NP, 128), jnp.float32)],
    )(a_tbl, acc_s0, acc_s1, efc_s0, efc_s1,
      acc_d0, acc_d1, efc_d0, efc_d1, w1e)


def _k5_l1_scatter(src_t, dst_t, src_g, dst_g, t_tbl, ps, pd, zero128):
    f32 = jnp.float32
    outs = (
        jax.ShapeDtypeStruct((2 * NP, 128), f32),
        jax.ShapeDtypeStruct((2 * NP, 128), f32),
    )

    @functools.partial(
        pl.kernel, mesh=_mesh(), out_type=outs,
        scratch_types=[
            pltpu.VMEM_SHARED((NP, 128), f32),
            pltpu.VMEM((EC, 128), f32),
            pltpu.VMEM((EC, 128), f32),
            pltpu.VMEM((EC,), jnp.int32),
            pltpu.VMEM((EC,), jnp.int32),
            pltpu.SemaphoreType.DMA,
        ],
    )
    def kern(src_t_h, dst_t_h, src_g_h, dst_g_h, t_h, ps_h, pd_h, z128_h,
             acc2_d_h, acc2_s_h,
             acc, rows_v, t_v, idx_t, idx_g, sem):
        c = lax.axis_index("c")
        s = lax.axis_index("s")
        wid = c * 16 + s
        base = wid * PER_W

        def one_pass(t_idx_h, g_idx_h, tbl_h, o_acc):
            pltpu.sync_copy(z128_h, rows_v)

            @pl.loop(0, ROWS_PER_TILE // EC)
            def _(k):
                r0 = s * ROWS_PER_TILE + k * EC
                pltpu.sync_copy(rows_v, acc.at[pl.ds(r0, EC)])
            plsc.subcore_barrier()

            @pl.loop(0, CHUNKS)
            def _(k):
                eb = base + k * EC
                pltpu.sync_copy(t_idx_h.at[pl.ds(eb, EC)], idx_t)
                pltpu.sync_copy(g_idx_h.at[pl.ds(eb, EC)], idx_g)
                pltpu.async_copy(tbl_h.at[idx_g], rows_v, sem).wait()
                pltpu.sync_copy(t_h.at[pl.ds(eb, EC)], t_v)

                @pl.loop(0, EC)
                def _(i):
                    for j in range(8):
                        sl = pl.ds(j * 16, 16)
                        v = rows_v[i, sl] + t_v[i, sl]
                        rows_v[i, sl] = jnp.maximum(v, 0.0)

                pltpu.sync_copy(rows_v, acc.at[idx_t], add=True)
            plsc.subcore_barrier()

            @pl.loop(0, ROWS_PER_TILE // EC)
            def _(k):
                r0 = s * ROWS_PER_TILE + k * EC
                pltpu.sync_copy(acc.at[pl.ds(r0, EC)], rows_v)
                pltpu.sync_copy(rows_v, o_acc.at[pl.ds(c * NP + r0, EC)])
            plsc.subcore_barrier()

        one_pass(dst_t_h, src_g_h, ps_h, acc2_d_h)
        one_pass(src_t_h, dst_g_h, pd_h, acc2_s_h)

    return kern(src_t, dst_t, src_g, dst_g, t_tbl, ps, pd, zero128)


def _k6_combine2(acc2_s0, acc2_s1, efc_s0, efc_s1,
                 acc2_d0, acc2_d1, efc_d0, efc_d1):
    def body(s0, s1, e0, e1, d0, d1, f0, f1, ms_ref, md_ref):
        cs = jnp.maximum((e0[...] + e1[...])[:, 16:17], 1.0)
        cd = jnp.maximum((f0[...] + f1[...])[:, 16:17], 1.0)
        ms_ref[...] = (s0[...] + s1[...]) / cs
        md_ref[...] = (d0[...] + d1[...]) / cd
    blk = NP // 8
    spec128 = pl.BlockSpec((blk, 128), lambda i: (i, 0))
    return pl.pallas_call(
        body,
        grid=(8,),
        in_specs=[spec128] * 8,
        out_specs=[spec128, spec128],
        out_shape=[jax.ShapeDtypeStruct((NP, 128), jnp.float32),
                   jax.ShapeDtypeStruct((NP, 128), jnp.float32)],
    )(acc2_s0, acc2_s1, efc_s0, efc_s1, acc2_d0, acc2_d1, efc_d0, efc_d1)


def _k7a_pos_gather(be, src16, dst16, ts16, t_tbl, ps, pd, ms, md):
    f32 = jnp.float32
    outs = (
        jax.ShapeDtypeStruct((B,), f32),
        jax.ShapeDtypeStruct((B, 128), f32),
        jax.ShapeDtypeStruct((B, 128), f32),
        jax.ShapeDtypeStruct((B, 128), f32),
        jax.ShapeDtypeStruct((B, 128), f32),
        jax.ShapeDtypeStruct((B, 128), f32),
    )
    QW = 32

    @functools.partial(
        pl.kernel, mesh=_mesh(), out_type=outs, compiler_params=_sc_cp(),
        scratch_types=[
            pltpu.VMEM((QW,), jnp.int32),
            pltpu.VMEM((QW,), jnp.int32),
            pltpu.VMEM((QW,), jnp.int32),
            pltpu.VMEM((QW,), f32),
            pltpu.VMEM((16, 128), jnp.int32),
            pltpu.VMEM((16, 128), f32),
            pltpu.VMEM((QW, 128), f32),
            pltpu.SemaphoreType.DMA,
        ],
    )
    def kern(be_h, src16_h, dst16_h, ts16_h, t_h, ps_h, pd_h, ms_h, md_h,
             tq_h, tbe_h, psb_h, msb_h, pdb_h, mdb_h,
             bev, bsv, bdv, tqv, irows, frows, rbuf, sem):
        c = lax.axis_index("c")
        s = lax.axis_index("s")
        wid = c * 16 + s

        @pl.when(wid < B // QW)
        def _():
            q0 = wid * QW
            pltpu.sync_copy(be_h.at[pl.ds(q0, QW)], bev)

            @pl.loop(0, QW // 16)
            def _(k):
                sl = pl.ds(k * 16, 16)
                ev = bev[sl]
                rows = lax.shift_right_logical(ev, 7)
                lanes = lax.bitwise_and(ev, 127)
                li = lax.iota(jnp.int32, 16)
                pltpu.async_copy(ts16_h.at[rows], frows, sem).wait()
                tqv[sl] = plsc.load_gather(frows, [li, lanes])
                pltpu.async_copy(src16_h.at[rows], irows, sem).wait()
                bsv[sl] = plsc.load_gather(irows, [li, lanes])
                pltpu.async_copy(dst16_h.at[rows], irows, sem).wait()
                bdv[sl] = plsc.load_gather(irows, [li, lanes])

            pltpu.sync_copy(tqv, tq_h.at[pl.ds(q0, QW)])

            def rows_out(tbl_h, idx_v, o_h):
                pltpu.async_copy(tbl_h.at[idx_v], rbuf, sem).wait()
                pltpu.sync_copy(rbuf, o_h.at[pl.ds(q0, QW)])

            rows_out(t_h, bev, tbe_h)
            rows_out(ps_h, bsv, psb_h)
            rows_out(ms_h, bsv, msb_h)
            rows_out(pd_h, bdv, pdb_h)
            rows_out(md_h, bdv, mdb_h)

    return kern(be, src16, dst16, ts16, t_tbl, ps, pd, ms, md)


def _k6b_neg_argmax(dst2d, ts2d, negf, tqcol):
    NCH = E // 128
    QG = 64

    def body(d_ref, t_ref, n_ref, q_ref, o_ref):
        lane = lax.broadcasted_iota(jnp.int32, (1, 128), 1).astype(jnp.float32)
        for g in range(B // QG):
            ng = n_ref[g * QG:(g + 1) * QG, :]
            qg = q_ref[g * QG:(g + 1) * QG, :]

            def step(ci, carry):
                bts, bidx = carry
                tsc = t_ref[pl.ds(ci, 1), :]
                dsc = d_ref[pl.ds(ci, 1), :]
                m = (dsc == ng) & (tsc <= qg)
                sc = jnp.where(m, tsc, -1e30)
                idxc = lane + ci.astype(jnp.float32) * 128.0
                upd = sc > bts
                bts = jnp.where(upd, sc, bts)
                bidx = jnp.where(upd, jnp.broadcast_to(idxc, bidx.shape),
                                 bidx)
                return bts, bidx

            init = (jnp.full((QG, 128), -3e38, jnp.float32),
                    jnp.zeros((QG, 128), jnp.float32))
            bts, bidx = lax.fori_loop(0, NCH, step, init)
            mx = jnp.max(bts, axis=1, keepdims=True)
            cand = jnp.where(bts >= mx, bidx, 3e38)
            o_ref[g * QG:(g + 1) * QG, :] = (
                jnp.min(cand, axis=1, keepdims=True).astype(jnp.int32))

    return pl.pallas_call(
        body,
        out_shape=jax.ShapeDtypeStruct((B, 1), jnp.int32),
    )(dst2d, ts2d, negf, tqcol)


def _k7b_neg_gather(ne, dst16, ts16, t_tbl, pd, md):
    f32 = jnp.float32
    outs = (
        jax.ShapeDtypeStruct((B,), f32),
        jax.ShapeDtypeStruct((B, 128), f32),
        jax.ShapeDtypeStruct((B, 128), f32),
        jax.ShapeDtypeStruct((B, 128), f32),
    )
    QW = 32

    @functools.partial(
        pl.kernel, mesh=_mesh(), out_type=outs, compiler_params=_sc_cp(),
        scratch_types=[
            pltpu.VMEM((QW,), jnp.int32),
            pltpu.VMEM((QW,), jnp.int32),
            pltpu.VMEM((QW,), f32),
            pltpu.VMEM((16, 128), jnp.int32),
            pltpu.VMEM((16, 128), f32),
            pltpu.VMEM((QW, 128), f32),
            pltpu.SemaphoreType.DMA,
        ],
    )
    def kern(ne_h, dst16_h, ts16_h, t_h, pd_h, md_h,
             tsn_h, tne_h, pdn_h, mdn_h,
             nev, ndv, tsv, irows, frows, rbuf, sem):
        c = lax.axis_index("c")
        s = lax.axis_index("s")
        wid = c * 16 + s

        @pl.when(wid < B // QW)
        def _():
            q0 = wid * QW
            pltpu.sync_copy(ne_h.at[pl.ds(q0, QW)], nev)

            @pl.loop(0, QW // 16)
            def _(k):
                sl = pl.ds(k * 16, 16)
                ev = nev[sl]
                rows = lax.shift_right_logical(ev, 7)
                lanes = lax.bitwise_and(ev, 127)
                li = lax.iota(jnp.int32, 16)
                pltpu.async_copy(ts16_h.at[rows], frows, sem).wait()
                tsv[sl] = plsc.load_gather(frows, [li, lanes])
                pltpu.async_copy(dst16_h.at[rows], irows, sem).wait()
                ndv[sl] = plsc.load_gather(irows, [li, lanes])

            pltpu.sync_copy(tsv, tsn_h.at[pl.ds(q0, QW)])

            def rows_out(tbl_h, idx_v, o_h):
                pltpu.async_copy(tbl_h.at[idx_v], rbuf, sem).wait()
                pltpu.sync_copy(rbuf, o_h.at[pl.ds(q0, QW)])

            rows_out(t_h, nev, tne_h)
            rows_out(pd_h, ndv, pdn_h)
            rows_out(md_h, ndv, mdn_h)

    return kern(ne, dst16, ts16, t_tbl, pd, md)


def _k8_head(tbe, psb, msb, pdb, mdb, tne, pdn, mdn, tqcol, tsncol,
             tw_row, w2a, w2b, w2c, gamma_row, beta_row, wp, bp_val):
    def body(tbe_r, psb_r, msb_r, pdb_r, mdb_r, tne_r, pdn_r, mdn_r,
             tq_r, tsn_r, tw_r, w2a_r, w2b_r, w2c_r, g_r, b_r, wp_r,
             bp_r, o_ref):
        tw = tw_r[...]
        w2a_ = w2a_r[...]
        w2b_ = w2b_r[...]
        w2c_ = w2c_r[...]
        tq = tq_r[...]
        tsn = tsn_r[...]

        def mm(x, w):
            return jnp.dot(x, w, preferred_element_type=jnp.float32)

        def layer2(x1, mrow, te):
            return jax.nn.relu(mm(x1, w2a_) + mm(mrow, w2b_) + mm(te, w2c_))

        def ln(x):
            m = jnp.mean(x, axis=1, keepdims=True)
            xc = x - m
            v = jnp.mean(xc * xc, axis=1, keepdims=True)
            return xc / jnp.sqrt(v + 1e-5) * g_r[...] + b_r[...]

        te_be = jnp.cos(tq * tw)
        te_ne = jnp.cos(tsn * tw)

        s1 = jax.nn.relu(psb_r[...] + tbe_r[...])
        d1 = jax.nn.relu(pdb_r[...] + tbe_r[...])
        n1 = jax.nn.relu(pdn_r[...] + tne_r[...])

        sf = ln(layer2(s1, msb_r[...], te_be))
        df = ln(layer2(d1, mdb_r[...], te_be))
        nf = ln(layer2(n1, mdn_r[...], te_ne))

        wp1 = wp_r[0:128, :]
        wp2 = wp_r[128:256, :]
        wp3 = wp_r[256:272, :]
        bp = bp_r[0, 0]
        u = mm(sf, wp1)
        pos = u + mm(df, wp2) + jnp.sum(wp3) + bp
        te_neg = jnp.cos((tq - tsn) * tw)
        neg = u + mm(nf, wp2) + mm(te_neg, wp3) + bp

        def softplus(z):
            return jnp.maximum(z, 0.0) + jnp.log1p(jnp.exp(-jnp.abs(z)))

        loss = jnp.mean(softplus(-pos)) + jnp.mean(softplus(neg))
        o_ref[...] = jnp.reshape(loss, (1, 1))

    return pl.pallas_call(
        body,
        out_shape=jax.ShapeDtypeStruct((1, 1), jnp.float32),
    )(tbe, psb, msb, pdb, mdb, tne, pdn, mdn, tqcol, tsncol,
      tw_row, w2a, w2b, w2c, gamma_row, beta_row, wp, bp_val)


def kernel(nfeat, efeat, edge_index, timestamp, batch_eids, neg_dst,
           t_w, W1, W2, gamma, beta, Wp, bp):
    f32 = jnp.float32
    i32 = jnp.int32
    src = edge_index[0].astype(i32)
    dst = edge_index[1].astype(i32)
    pad_e = EP - E

    nfeat_p = jnp.pad(nfeat, ((0, NP - N), (0, 0)))
    src_t = jnp.concatenate([src, jnp.full((pad_e,), N, i32)])
    dst_t = jnp.concatenate([dst, jnp.full((pad_e,), N, i32)])
    src_g = jnp.concatenate([src, jnp.zeros((pad_e,), i32)])
    dst_g = jnp.concatenate([dst, jnp.zeros((pad_e,), i32)])
    efeat_p = jnp.pad(efeat, ((0, pad_e), (0, 0)))
    ts_p = jnp.pad(timestamp, (0, pad_e))

    w1cat = jnp.concatenate([W1[0:128], W1[144:272]], axis=1)
    wef = W1[128:144]
    w1e = W1[272:288]
    wtf = W1[288:304]
    tw_row = t_w.reshape(1, TD)

    zero128 = jnp.zeros((EC, 128), f32)
    efw = jnp.concatenate(
        [efeat, jnp.ones((E, 1), f32), jnp.zeros((E, 111), f32)], axis=1)
    efw = jnp.pad(efw, ((0, pad_e), (0, 0)))

    ab = _k1_node_mm(nfeat_p, w1cat)
    a_tbl = ab[:, 0:128]
    bt = ab[:, 128:256]

    t_tbl = _k2_edge_t(ts_p.reshape(EP, 1), efeat_p, wef, wtf, tw_row)

    acc_s, efc_s, acc_d, efc_d = _k3_l0_scatter(
        src_t, dst_t, src_g, dst_g, efw, bt, zero128)

    ps, pd = _k4_combine1(
        a_tbl, acc_s[:NP], acc_s[NP:], efc_s[:NP], efc_s[NP:],
        acc_d[:NP], acc_d[NP:], efc_d[:NP], efc_d[NP:], w1e)

    acc2_d, acc2_s = _k5_l1_scatter(
        src_t, dst_t, src_g, dst_g, t_tbl, ps, pd, zero128)

    ms, md = _k6_combine2(
        acc2_s[:NP], acc2_s[NP:], efc_s[:NP], efc_s[NP:],
        acc2_d[:NP], acc2_d[NP:], efc_d[:NP], efc_d[NP:])

    src16 = src_g.reshape(EP // 128, 128)
    dst16 = dst_g.reshape(EP // 128, 128)
    ts16 = ts_p.reshape(EP // 128, 128)

    tq, tbe, psb, msb, pdb, mdb = _k7a_pos_gather(
        batch_eids.astype(i32), src16, dst16, ts16, t_tbl, ps, pd, ms, md)

    ne = _k6b_neg_argmax(
        dst.astype(f32).reshape(E // 128, 128),
        timestamp.reshape(E // 128, 128),
        neg_dst.astype(f32).reshape(B, 1),
        tq.reshape(B, 1))

    tsn, tne, pdn, mdn = _k7b_neg_gather(
        ne.reshape(B), dst16, ts16, t_tbl, pd, md)

    loss = _k8_head(
        tbe, psb, msb, pdb, mdb, tne, pdn, mdn,
        tq.reshape(B, 1), tsn.reshape(B, 1), tw_row,
        W2[0:128], W2[128:256], W2[256:272],
        gamma.reshape(1, H), beta.reshape(1, H), Wp, bp.reshape(1, 1))

    return loss[0, 0]


# speedup vs baseline: 2.0948x; 1.3128x over previous
import dataclasses
import functools

import jax
import jax.numpy as jnp
from jax import lax
from jax.experimental import pallas as pl
from jax.experimental.pallas import tpu as pltpu
from jax.experimental.pallas import tpu_sc as plsc

N = 10000
NP = 10240
E = 160000
EP = 163840
DE = 16
TD = 16
H = 128
B = 256

NW = 32
EC = 64
PER_W = EP // NW
CHUNKS = PER_W // EC
ROWS_PER_TILE = NP // 16

@functools.cache
def _mesh():
    return plsc.VectorSubcoreMesh(core_axis_name="c", subcore_axis_name="s",
                                  num_cores=2, num_subcores=16)


def _sc_cp():
    cp = pltpu.CompilerParams()
    if "needs_layout_passes" in pltpu.CompilerParams.__dataclass_fields__:
        cp = dataclasses.replace(cp, needs_layout_passes=False)
    return cp


def _k1_node_mm(nfeat_p, wcat):
    def body(x_ref, w_ref, o_ref):
        o_ref[...] = jnp.dot(x_ref[...], w_ref[...],
                             preferred_element_type=jnp.float32)
    return pl.pallas_call(
        body,
        grid=(8,),
        in_specs=[pl.BlockSpec((NP // 8, 128), lambda i: (i, 0)),
                  pl.BlockSpec((128, 256), lambda i: (0, 0))],
        out_specs=pl.BlockSpec((NP // 8, 256), lambda i: (i, 0)),
        out_shape=jax.ShapeDtypeStruct((NP, 256), jnp.float32),
    )(nfeat_p, wcat)


def _k2_edge_t(ts_col, efeat_p, wef, wtf, tw_row):
    def body(ts_ref, ef_ref, wef_ref, wtf_ref, tw_ref, o_ref):
        te = jnp.cos(ts_ref[...] * tw_ref[...])
        o_ref[...] = (
            jnp.dot(ef_ref[...], wef_ref[...],
                    preferred_element_type=jnp.float32)
            + jnp.dot(te, wtf_ref[...],
                      preferred_element_type=jnp.float32))
    blk = 2048
    return pl.pallas_call(
        body,
        grid=(EP // blk,),
        in_specs=[pl.BlockSpec((blk, 1), lambda i: (i, 0)),
                  pl.BlockSpec((blk, 16), lambda i: (i, 0)),
                  pl.BlockSpec((16, 128), lambda i: (0, 0)),
                  pl.BlockSpec((16, 128), lambda i: (0, 0)),
                  pl.BlockSpec((1, 16), lambda i: (0, 0))],
        out_specs=pl.BlockSpec((blk, 128), lambda i: (i, 0)),
        out_shape=jax.ShapeDtypeStruct((EP, 128), jnp.float32),
    )(ts_col, efeat_p, wef, wtf, tw_row)


def _k3_l0_scatter(src_t3, dst_t3, src_g3, dst_g3, efw, bt, zero128):
    f32 = jnp.float32
    outs = (
        jax.ShapeDtypeStruct((2 * NP, 128), f32),
        jax.ShapeDtypeStruct((2 * NP, 128), f32),
        jax.ShapeDtypeStruct((2 * NP, 128), f32),
        jax.ShapeDtypeStruct((2 * NP, 128), f32),
    )

    @functools.partial(
        pl.kernel, mesh=_mesh(), out_type=outs,
        scratch_types=[
            pltpu.VMEM_SHARED((NP, 128), f32),
            pltpu.VMEM((EC, 128), f32),
            pltpu.VMEM((EC, 128), f32),
            pltpu.VMEM((CHUNKS, EC), jnp.int32),
            pltpu.VMEM((CHUNKS, EC), jnp.int32),
            pltpu.SemaphoreType.DMA,
            pltpu.SemaphoreType.DMA,
        ],
    )
    def kern(src_t_h, dst_t_h, src_g_h, dst_g_h, efw_h, bt_h, z128_h,
             acc_s_h, efc_s_h, acc_d_h, efc_d_h,
             sh, buf0, buf1, idx_t, idx_g, sem0, sem1):
        c = lax.axis_index("c")
        s = lax.axis_index("s")
        wid = c * 16 + s
        base = wid * PER_W
        bufs = (buf0, buf1)
        sems = (sem0, sem1)

        def zero_and_sync():
            pltpu.sync_copy(z128_h, buf0)

            @pl.loop(0, ROWS_PER_TILE // EC)
            def _(k):
                r0 = s * ROWS_PER_TILE + k * EC
                pltpu.sync_copy(buf0, sh.at[pl.ds(r0, EC)])
            plsc.subcore_barrier()

        def writeback(o_h):
            plsc.subcore_barrier()

            @pl.loop(0, ROWS_PER_TILE // EC)
            def _(k):
                r0 = s * ROWS_PER_TILE + k * EC
                pltpu.sync_copy(sh.at[pl.ds(r0, EC)], buf0)
                pltpu.sync_copy(buf0, o_h.at[pl.ds(c * NP + r0, EC)])
            plsc.subcore_barrier()

        def gsrc(tbl_h, kk, gather):
            if gather:
                return tbl_h.at[idx_g.at[kk]]
            return tbl_h.at[pl.ds(base + kk * EC, EC)]

        def scatter_pass(tbl_h, o_h, gather):
            zero_and_sync()
            pltpu.async_copy(gsrc(tbl_h, 0, gather), buf0, sem0)

            @pl.loop(0, CHUNKS, step=2)
            def _(k):
                for b in range(2):
                    kk = k + b
                    pltpu.make_async_copy(
                        gsrc(tbl_h, 0, gather), bufs[b], sems[b]).wait()

                    @pl.when(kk + 1 < CHUNKS)
                    def _():
                        pltpu.async_copy(
                            gsrc(tbl_h, kk + 1, gather),
                            bufs[1 - b], sems[1 - b])

                    pltpu.sync_copy(bufs[b], sh.at[idx_t.at[kk]], add=True)
            writeback(o_h)

        pltpu.sync_copy(src_t_h.at[pl.ds(wid * CHUNKS, CHUNKS)], idx_t)
        pltpu.sync_copy(dst_g_h.at[pl.ds(wid * CHUNKS, CHUNKS)], idx_g)
        scatter_pass(bt_h, acc_s_h, gather=True)
        scatter_pass(efw_h, efc_s_h, gather=False)
        pltpu.sync_copy(dst_t_h.at[pl.ds(wid * CHUNKS, CHUNKS)], idx_t)
        pltpu.sync_copy(src_g_h.at[pl.ds(wid * CHUNKS, CHUNKS)], idx_g)
        scatter_pass(bt_h, acc_d_h, gather=True)
        scatter_pass(efw_h, efc_d_h, gather=False)

    return kern(src_t3, dst_t3, src_g3, dst_g3, efw, bt, zero128)


def _k4_combine1(a_tbl, acc_s0, acc_s1, efc_s0, efc_s1,
                 acc_d0, acc_d1, efc_d0, efc_d1, w1e):
    def body(a_ref, s0, s1, e0, e1, d0, d1, f0, f1, w_ref, ps_ref, pd_ref):
        a = a_ref[...]
        w = w_ref[...]
        es = e0[...] + e1[...]
        ed = f0[...] + f1[...]
        cs = jnp.maximum(es[:, 16:17], 1.0)
        cd = jnp.maximum(ed[:, 16:17], 1.0)
        num_s = (s0[...] + s1[...]) + jnp.dot(
            es[:, 0:16], w, preferred_element_type=jnp.float32)
        num_d = (d0[...] + d1[...]) + jnp.dot(
            ed[:, 0:16], w, preferred_element_type=jnp.float32)
        ps_ref[...] = a + num_s / cs
        pd_ref[...] = a + num_d / cd
    blk = NP // 8
    spec128 = pl.BlockSpec((blk, 128), lambda i: (i, 0))
    return pl.pallas_call(
        body,
        grid=(8,),
        in_specs=[spec128] * 9 + [pl.BlockSpec((16, 128), lambda i: (0, 0))],
        out_specs=[spec128, spec128],
        out_shape=[jax.ShapeDtypeStruct((NP, 128), jnp.float32),
                   jax.ShapeDtypeStruct((NP, 128), jnp.float32)],
    )(a_tbl, acc_s0, acc_s1, efc_s0, efc_s1,
      acc_d0, acc_d1, efc_d0, efc_d1, w1e)


def _k5_l1_scatter(src_t3, dst_t3, src_g3, dst_g3, t_tbl, ps, pd, zero128):
    f32 = jnp.float32
    outs = (
        jax.ShapeDtypeStruct((2 * NP, 128), f32),
        jax.ShapeDtypeStruct((2 * NP, 128), f32),
    )

    @functools.partial(
        pl.kernel, mesh=_mesh(), out_type=outs,
        scratch_types=[
            pltpu.VMEM_SHARED((NP, 128), f32),
            pltpu.VMEM((EC, 128), f32),
            pltpu.VMEM((EC, 128), f32),
            pltpu.VMEM((EC, 128), f32),
            pltpu.VMEM((EC, 128), f32),
            pltpu.VMEM((CHUNKS // 2, EC), jnp.int32),
            pltpu.VMEM((CHUNKS // 2, EC), jnp.int32),
            pltpu.SemaphoreType.DMA,
            pltpu.SemaphoreType.DMA,
            pltpu.SemaphoreType.DMA,
            pltpu.SemaphoreType.DMA,
        ],
    )
    def kern(src_t_h, dst_t_h, src_g_h, dst_g_h, t_h, ps_h, pd_h, z128_h,
             acc2_d_h, acc2_s_h,
             sh, buf0, buf1, tb0, tb1, idx_t, idx_g,
             sem0, sem1, tsem0, tsem1):
        c = lax.axis_index("c")
        s = lax.axis_index("s")
        wid = c * 16 + s
        base = wid * PER_W
        bufs = (buf0, buf1)
        tbufs = (tb0, tb1)
        sems = (sem0, sem1)
        tsems = (tsem0, tsem1)

        def zero_and_sync():
            pltpu.sync_copy(z128_h, buf0)

            @pl.loop(0, ROWS_PER_TILE // EC)
            def _(k):
                r0 = s * ROWS_PER_TILE + k * EC
                pltpu.sync_copy(buf0, sh.at[pl.ds(r0, EC)])
            plsc.subcore_barrier()

        def writeback(o_h):
            plsc.subcore_barrier()

            @pl.loop(0, ROWS_PER_TILE // EC)
            def _(k):
                r0 = s * ROWS_PER_TILE + k * EC
                pltpu.sync_copy(sh.at[pl.ds(r0, EC)], buf0)
                pltpu.sync_copy(buf0, o_h.at[pl.ds(c * NP + r0, EC)])
            plsc.subcore_barrier()

        HC = CHUNKS // 2

        def one_pass(t_idx_h, g_idx_h, tbl_h, o_h):
            zero_and_sync()
            for h in range(2):
                hb = base + h * HC * EC
                pltpu.sync_copy(
                    t_idx_h.at[pl.ds(wid * CHUNKS + h * HC, HC)], idx_t)
                pltpu.sync_copy(
                    g_idx_h.at[pl.ds(wid * CHUNKS + h * HC, HC)], idx_g)
                pltpu.async_copy(tbl_h.at[idx_g.at[0]], buf0, sem0)
                pltpu.async_copy(t_h.at[pl.ds(hb, EC)], tb0, tsem0)

                @pl.loop(0, HC, step=2)
                def _(k):
                    for b in range(2):
                        kk = k + b
                        pltpu.make_async_copy(
                            tbl_h.at[idx_g.at[0]], bufs[b], sems[b]).wait()
                        pltpu.make_async_copy(
                            t_h.at[pl.ds(hb, EC)], tbufs[b], tsems[b]).wait()

                        @pl.when(kk + 1 < HC)
                        def _():
                            pltpu.async_copy(tbl_h.at[idx_g.at[kk + 1]],
                                             bufs[1 - b], sems[1 - b])
                            pltpu.async_copy(
                                t_h.at[pl.ds(hb + (kk + 1) * EC, EC)],
                                tbufs[1 - b], tsems[1 - b])

                        @pl.loop(0, EC)
                        def _(i):
                            for j in range(8):
                                sl = pl.ds(j * 16, 16)
                                v = bufs[b][i, sl] + tbufs[b][i, sl]
                                bufs[b][i, sl] = jnp.maximum(v, 0.0)

                        pltpu.sync_copy(bufs[b], sh.at[idx_t.at[kk]],
                                        add=True)
            writeback(o_h)

        one_pass(dst_t_h, src_g_h, ps_h, acc2_d_h)
        one_pass(src_t_h, dst_g_h, pd_h, acc2_s_h)

    return kern(src_t3, dst_t3, src_g3, dst_g3, t_tbl, ps, pd, zero128)


def _k6_combine2(acc2_s0, acc2_s1, efc_s0, efc_s1,
                 acc2_d0, acc2_d1, efc_d0, efc_d1):
    def body(s0, s1, e0, e1, d0, d1, f0, f1, ms_ref, md_ref):
        cs = jnp.maximum((e0[...] + e1[...])[:, 16:17], 1.0)
        cd = jnp.maximum((f0[...] + f1[...])[:, 16:17], 1.0)
        ms_ref[...] = (s0[...] + s1[...]) / cs
        md_ref[...] = (d0[...] + d1[...]) / cd
    blk = NP // 8
    spec128 = pl.BlockSpec((blk, 128), lambda i: (i, 0))
    return pl.pallas_call(
        body,
        grid=(8,),
        in_specs=[spec128] * 8,
        out_specs=[spec128, spec128],
        out_shape=[jax.ShapeDtypeStruct((NP, 128), jnp.float32),
                   jax.ShapeDtypeStruct((NP, 128), jnp.float32)],
    )(acc2_s0, acc2_s1, efc_s0, efc_s1, acc2_d0, acc2_d1, efc_d0, efc_d1)


def _k7a_pos_gather(be, src16, dst16, ts16, t_tbl, ps, pd, ms, md):
    f32 = jnp.float32
    outs = (
        jax.ShapeDtypeStruct((B,), f32),
        jax.ShapeDtypeStruct((B, 128), f32),
        jax.ShapeDtypeStruct((B, 128), f32),
        jax.ShapeDtypeStruct((B, 128), f32),
        jax.ShapeDtypeStruct((B, 128), f32),
        jax.ShapeDtypeStruct((B, 128), f32),
    )
    QW = 32

    @functools.partial(
        pl.kernel, mesh=_mesh(), out_type=outs, compiler_params=_sc_cp(),
        scratch_types=[
            pltpu.VMEM((QW,), jnp.int32),
            pltpu.VMEM((QW,), jnp.int32),
            pltpu.VMEM((QW,), jnp.int32),
            pltpu.VMEM((QW,), f32),
            pltpu.VMEM((16, 128), jnp.int32),
            pltpu.VMEM((16, 128), f32),
            pltpu.VMEM((QW, 128), f32),
            pltpu.SemaphoreType.DMA,
        ],
    )
    def kern(be_h, src16_h, dst16_h, ts16_h, t_h, ps_h, pd_h, ms_h, md_h,
             tq_h, tbe_h, psb_h, msb_h, pdb_h, mdb_h,
             bev, bsv, bdv, tqv, irows, frows, rbuf, sem):
        c = lax.axis_index("c")
        s = lax.axis_index("s")
        wid = c * 16 + s

        @pl.when(wid < B // QW)
        def _():
            q0 = wid * QW
            pltpu.sync_copy(be_h.at[pl.ds(q0, QW)], bev)

            @pl.loop(0, QW // 16)
            def _(k):
                sl = pl.ds(k * 16, 16)
                ev = bev[sl]
                rows = lax.shift_right_logical(ev, 7)
                lanes = lax.bitwise_and(ev, 127)
                li = lax.iota(jnp.int32, 16)
                pltpu.async_copy(ts16_h.at[rows], frows, sem).wait()
                tqv[sl] = plsc.load_gather(frows, [li, lanes])
                pltpu.async_copy(src16_h.at[rows], irows, sem).wait()
                bsv[sl] = plsc.load_gather(irows, [li, lanes])
                pltpu.async_copy(dst16_h.at[rows], irows, sem).wait()
                bdv[sl] = plsc.load_gather(irows, [li, lanes])

            pltpu.sync_copy(tqv, tq_h.at[pl.ds(q0, QW)])

            def rows_out(tbl_h, idx_v, o_h):
                pltpu.async_copy(tbl_h.at[idx_v], rbuf, sem).wait()
                pltpu.sync_copy(rbuf, o_h.at[pl.ds(q0, QW)])

            rows_out(t_h, bev, tbe_h)
            rows_out(ps_h, bsv, psb_h)
            rows_out(ms_h, bsv, msb_h)
            rows_out(pd_h, bdv, pdb_h)
            rows_out(md_h, bdv, mdb_h)

    return kern(be, src16, dst16, ts16, t_tbl, ps, pd, ms, md)


def _k6b_neg_argmax(dst2d, ts2d, negf, tqcol):
    NCH = E // 128
    QG = 64

    def body(d_ref, t_ref, n_ref, q_ref, o_ref):
        lane = lax.broadcasted_iota(jnp.int32, (1, 128), 1).astype(jnp.float32)
        for g in range(B // QG):
            ng = n_ref[g * QG:(g + 1) * QG, :]
            qg = q_ref[g * QG:(g + 1) * QG, :]

            def step(ci, carry):
                bts, bidx = carry
                tsc = t_ref[pl.ds(ci, 1), :]
                dsc = d_ref[pl.ds(ci, 1), :]
                m = (dsc == ng) & (tsc <= qg)
                sc = jnp.where(m, tsc, -1e30)
                idxc = lane + ci.astype(jnp.float32) * 128.0
                upd = sc > bts
                bts = jnp.where(upd, sc, bts)
                bidx = jnp.where(upd, jnp.broadcast_to(idxc, bidx.shape),
                                 bidx)
                return bts, bidx

            init = (jnp.full((QG, 128), -3e38, jnp.float32),
                    jnp.zeros((QG, 128), jnp.float32))
            bts, bidx = lax.fori_loop(0, NCH, step, init)
            mx = jnp.max(bts, axis=1, keepdims=True)
            cand = jnp.where(bts >= mx, bidx, 3e38)
            o_ref[g * QG:(g + 1) * QG, :] = (
                jnp.min(cand, axis=1, keepdims=True).astype(jnp.int32))

    return pl.pallas_call(
        body,
        out_shape=jax.ShapeDtypeStruct((B, 1), jnp.int32),
    )(dst2d, ts2d, negf, tqcol)


def _k7b_neg_gather(ne, dst16, ts16, t_tbl, pd, md):
    f32 = jnp.float32
    outs = (
        jax.ShapeDtypeStruct((B,), f32),
        jax.ShapeDtypeStruct((B, 128), f32),
        jax.ShapeDtypeStruct((B, 128), f32),
        jax.ShapeDtypeStruct((B, 128), f32),
    )
    QW = 32

    @functools.partial(
        pl.kernel, mesh=_mesh(), out_type=outs, compiler_params=_sc_cp(),
        scratch_types=[
            pltpu.VMEM((QW,), jnp.int32),
            pltpu.VMEM((QW,), jnp.int32),
            pltpu.VMEM((QW,), f32),
            pltpu.VMEM((16, 128), jnp.int32),
            pltpu.VMEM((16, 128), f32),
            pltpu.VMEM((QW, 128), f32),
            pltpu.SemaphoreType.DMA,
        ],
    )
    def kern(ne_h, dst16_h, ts16_h, t_h, pd_h, md_h,
             tsn_h, tne_h, pdn_h, mdn_h,
             nev, ndv, tsv, irows, frows, rbuf, sem):
        c = lax.axis_index("c")
        s = lax.axis_index("s")
        wid = c * 16 + s

        @pl.when(wid < B // QW)
        def _():
            q0 = wid * QW
            pltpu.sync_copy(ne_h.at[pl.ds(q0, QW)], nev)

            @pl.loop(0, QW // 16)
            def _(k):
                sl = pl.ds(k * 16, 16)
                ev = nev[sl]
                rows = lax.shift_right_logical(ev, 7)
                lanes = lax.bitwise_and(ev, 127)
                li = lax.iota(jnp.int32, 16)
                pltpu.async_copy(ts16_h.at[rows], frows, sem).wait()
                tsv[sl] = plsc.load_gather(frows, [li, lanes])
                pltpu.async_copy(dst16_h.at[rows], irows, sem).wait()
                ndv[sl] = plsc.load_gather(irows, [li, lanes])

            pltpu.sync_copy(tsv, tsn_h.at[pl.ds(q0, QW)])

            def rows_out(tbl_h, idx_v, o_h):
                pltpu.async_copy(tbl_h.at[idx_v], rbuf, sem).wait()
                pltpu.sync_copy(rbuf, o_h.at[pl.ds(q0, QW)])

            rows_out(t_h, nev, tne_h)
            rows_out(pd_h, ndv, pdn_h)
            rows_out(md_h, ndv, mdn_h)

    return kern(ne, dst16, ts16, t_tbl, pd, md)


def _k8_head(tbe, psb, msb, pdb, mdb, tne, pdn, mdn, tqcol, tsncol,
             tw_row, w2a, w2b, w2c, gamma_row, beta_row, wp, bp_val):
    def body(tbe_r, psb_r, msb_r, pdb_r, mdb_r, tne_r, pdn_r, mdn_r,
             tq_r, tsn_r, tw_r, w2a_r, w2b_r, w2c_r, g_r, b_r, wp_r,
             bp_r, o_ref):
        tw = tw_r[...]
        w2a_ = w2a_r[...]
        w2b_ = w2b_r[...]
        w2c_ = w2c_r[...]
        tq = tq_r[...]
        tsn = tsn_r[...]

        def mm(x, w):
            return jnp.dot(x, w, preferred_element_type=jnp.float32)

        def layer2(x1, mrow, te):
            return jax.nn.relu(mm(x1, w2a_) + mm(mrow, w2b_) + mm(te, w2c_))

        def ln(x):
            m = jnp.mean(x, axis=1, keepdims=True)
            xc = x - m
            v = jnp.mean(xc * xc, axis=1, keepdims=True)
            return xc / jnp.sqrt(v + 1e-5) * g_r[...] + b_r[...]

        te_be = jnp.cos(tq * tw)
        te_ne = jnp.cos(tsn * tw)

        s1 = jax.nn.relu(psb_r[...] + tbe_r[...])
        d1 = jax.nn.relu(pdb_r[...] + tbe_r[...])
        n1 = jax.nn.relu(pdn_r[...] + tne_r[...])

        sf = ln(layer2(s1, msb_r[...], te_be))
        df = ln(layer2(d1, mdb_r[...], te_be))
        nf = ln(layer2(n1, mdn_r[...], te_ne))

        wp1 = wp_r[0:128, :]
        wp2 = wp_r[128:256, :]
        wp3 = wp_r[256:272, :]
        bp = bp_r[0, 0]
        u = mm(sf, wp1)
        pos = u + mm(df, wp2) + jnp.sum(wp3) + bp
        te_neg = jnp.cos((tq - tsn) * tw)
        neg = u + mm(nf, wp2) + mm(te_neg, wp3) + bp

        def softplus(z):
            return jnp.maximum(z, 0.0) + jnp.log1p(jnp.exp(-jnp.abs(z)))

        loss = jnp.mean(softplus(-pos)) + jnp.mean(softplus(neg))
        o_ref[...] = jnp.reshape(loss, (1, 1))

    return pl.pallas_call(
        body,
        out_shape=jax.ShapeDtypeStruct((1, 1), jnp.float32),
    )(tbe, psb, msb, pdb, mdb, tne, pdn, mdn, tqcol, tsncol,
      tw_row, w2a, w2b, w2c, gamma_row, beta_row, wp, bp_val)


def kernel(nfeat, efeat, edge_index, timestamp, batch_eids, neg_dst,
           t_w, W1, W2, gamma, beta, Wp, bp):
    f32 = jnp.float32
    i32 = jnp.int32
    src = edge_index[0].astype(i32)
    dst = edge_index[1].astype(i32)
    pad_e = EP - E

    nfeat_p = jnp.pad(nfeat, ((0, NP - N), (0, 0)))
    src_t = jnp.concatenate([src, jnp.full((pad_e,), N, i32)])
    dst_t = jnp.concatenate([dst, jnp.full((pad_e,), N, i32)])
    src_g = jnp.concatenate([src, jnp.zeros((pad_e,), i32)])
    dst_g = jnp.concatenate([dst, jnp.zeros((pad_e,), i32)])
    efeat_p = jnp.pad(efeat, ((0, pad_e), (0, 0)))
    ts_p = jnp.pad(timestamp, (0, pad_e))

    w1cat = jnp.concatenate([W1[0:128], W1[144:272]], axis=1)
    wef = W1[128:144]
    w1e = W1[272:288]
    wtf = W1[288:304]
    tw_row = t_w.reshape(1, TD)

    zero128 = jnp.zeros((EC, 128), f32)
    efw = jnp.concatenate(
        [efeat, jnp.ones((E, 1), f32), jnp.zeros((E, 111), f32)], axis=1)
    efw = jnp.pad(efw, ((0, pad_e), (0, 0)))

    ab = _k1_node_mm(nfeat_p, w1cat)
    a_tbl = ab[:, 0:128]
    bt = ab[:, 128:256]

    t_tbl = _k2_edge_t(ts_p.reshape(EP, 1), efeat_p, wef, wtf, tw_row)

    src_t3 = src_t.reshape(NW * CHUNKS, EC)
    dst_t3 = dst_t.reshape(NW * CHUNKS, EC)
    src_g3 = src_g.reshape(NW * CHUNKS, EC)
    dst_g3 = dst_g.reshape(NW * CHUNKS, EC)

    acc_s, efc_s, acc_d, efc_d = _k3_l0_scatter(
        src_t3, dst_t3, src_g3, dst_g3, efw, bt, zero128)

    ps, pd = _k4_combine1(
        a_tbl, acc_s[:NP], acc_s[NP:], efc_s[:NP], efc_s[NP:],
        acc_d[:NP], acc_d[NP:], efc_d[:NP], efc_d[NP:], w1e)

    acc2_d, acc2_s = _k5_l1_scatter(
        src_t3, dst_t3, src_g3, dst_g3, t_tbl, ps, pd, zero128)

    ms, md = _k6_combine2(
        acc2_s[:NP], acc2_s[NP:], efc_s[:NP], efc_s[NP:],
        acc2_d[:NP], acc2_d[NP:], efc_d[:NP], efc_d[NP:])

    src16 = src_g.reshape(EP // 128, 128)
    dst16 = dst_g.reshape(EP // 128, 128)
    ts16 = ts_p.reshape(EP // 128, 128)

    tq, tbe, psb, msb, pdb, mdb = _k7a_pos_gather(
        batch_eids.astype(i32), src16, dst16, ts16, t_tbl, ps, pd, ms, md)

    ne = _k6b_neg_argmax(
        dst.astype(f32).reshape(E // 128, 128),
        timestamp.reshape(E // 128, 128),
        neg_dst.astype(f32).reshape(B, 1),
        tq.reshape(B, 1))

    tsn, tne, pdn, mdn = _k7b_neg_gather(
        ne.reshape(B), dst16, ts16, t_tbl, pd, md)

    loss = _k8_head(
        tbe, psb, msb, pdb, mdb, tne, pdn, mdn,
        tq.reshape(B, 1), tsn.reshape(B, 1), tw_row,
        W2[0:128], W2[128:256], W2[256:272],
        gamma.reshape(1, H), beta.reshape(1, H), Wp, bp.reshape(1, 1))

    return loss[0, 0]
